# Initial kernel scaffold; baseline (speedup 1.0000x reference)
import jax
import jax.numpy as jnp
from jax.experimental import pallas as pl


def kernel(x, edge_index, W1l, b1, W1r, W2l, b2, W2r, W3l, b3, W3r, Wc, bc, a):
    raise NotImplementedError("write your pallas kernel here")


# speedup vs baseline: 7.5665x; 7.5665x over previous
import functools

import jax
import jax.numpy as jnp
from jax import lax
from jax.experimental import pallas as pl
from jax.experimental.pallas import tpu as pltpu
from jax.experimental.pallas import tpu_sc as plsc

N = 10000
E = 320000
NC, NS = 2, 16
NW = NC * NS
CH = 128
EPT = 10240
NCHUNK = EPT // CH
ACC = 10240
RPT = ACC // NS
EPAD = NW * EPT - E

_MESH = dict(core_axis_name="c", subcore_axis_name="s",
             num_cores=NC, num_subcores=NS)


def _make_seg_sum(d):
  mesh = plsc.VectorSubcoreMesh(**_MESH)
  scratch = [
      pltpu.VMEM((NCHUNK, CH), jnp.int32),
      pltpu.VMEM((NCHUNK, CH), jnp.int32),
      pltpu.VMEM((CH, d), jnp.float32),
      pltpu.VMEM_SHARED((ACC, d), jnp.float32),
      pltpu.SemaphoreType.DMA,
  ]

  def body(m_hbm, src_hbm, dst_hbm, zrow_hbm, out_hbm,
           sidx, didx, buf, acc, sem):
    c = lax.axis_index("c")
    s = lax.axis_index("s")
    wid = c * NS + s

    pltpu.sync_copy(src_hbm.at[wid], sidx)
    pltpu.sync_copy(dst_hbm.at[wid], didx)

    pltpu.sync_copy(zrow_hbm, buf)
    for k in range(RPT // CH):
      pltpu.sync_copy(buf, acc.at[pl.ds(s * RPT + k * CH, CH)])
    plsc.subcore_barrier()

    @pl.loop(0, NCHUNK)
    def _(j):
      pltpu.async_copy(m_hbm.at[sidx.at[j]], buf, sem).wait()
      pltpu.sync_copy(buf, acc.at[didx.at[j]], add=True)

    plsc.subcore_barrier()
    pltpu.sync_copy(acc.at[pl.ds(s * RPT, RPT)],
                    out_hbm.at[c, pl.ds(s * RPT, RPT)])

  return pl.kernel(body,
                   out_type=jax.ShapeDtypeStruct((NC, ACC, d), jnp.float32),
                   mesh=mesh, scratch_types=scratch)


CW = 128


def _make_counts():
  mesh = plsc.VectorSubcoreMesh(**_MESH)
  scratch = [
      pltpu.VMEM((NCHUNK, CH), jnp.int32),
      pltpu.VMEM((CH, CW), jnp.float32),
      pltpu.VMEM_SHARED((ACC, CW), jnp.float32),
  ]

  def body(dst_hbm, ones_hbm, zrow_hbm, cnt_hbm, didx, buf, cacc):
    c = lax.axis_index("c")
    s = lax.axis_index("s")
    wid = c * NS + s
    pltpu.sync_copy(dst_hbm.at[wid], didx)
    pltpu.sync_copy(zrow_hbm, buf)
    for k in range(RPT // CH):
      pltpu.sync_copy(buf, cacc.at[pl.ds(s * RPT + k * CH, CH)])
    pltpu.sync_copy(ones_hbm, buf)
    plsc.subcore_barrier()

    @pl.loop(0, NCHUNK)
    def _(j):
      pltpu.sync_copy(buf, cacc.at[didx.at[j]], add=True)

    plsc.subcore_barrier()
    pltpu.sync_copy(cacc.at[pl.ds(s * RPT, RPT)],
                    cnt_hbm.at[c, pl.ds(s * RPT, RPT)])

  return pl.kernel(body,
                   out_type=jax.ShapeDtypeStruct((NC, ACC, CW), jnp.float32),
                   mesh=mesh, scratch_types=scratch)


_seg_sum_128 = _make_seg_sum(128)
_counts = _make_counts()

_ROWS = 400
_GRID = N // _ROWS


def _tc_layer1(s_ref, c_ref, x_ref, wl_ref, wr_ref, b_ref, a_ref, o_ref):
  cnt = jnp.maximum(c_ref[0][:, 0:1] + c_ref[1][:, 0:1], 1.0)
  mean = (s_ref[0] + s_ref[1]) / cnt
  z = (jnp.dot(mean, wl_ref[...], preferred_element_type=jnp.float32)
       + jnp.dot(x_ref[...], wr_ref[...], preferred_element_type=jnp.float32)
       + b_ref[...])
  o_ref[...] = jnp.where(z >= 0.0, z, a_ref[0, 0] * z)


def _tc_layer2(s_ref, c_ref, h_ref, wl_ref, wr_ref, b_ref, a_ref, o_ref):
  cnt = jnp.maximum(c_ref[0][:, 0:1] + c_ref[1][:, 0:1], 1.0)
  mean = (s_ref[0] + s_ref[1]) / cnt
  z = (jnp.dot(mean, wl_ref[...], preferred_element_type=jnp.float32)
       + jnp.dot(h_ref[...], wr_ref[...], preferred_element_type=jnp.float32)
       + b_ref[...])
  o_ref[...] = jnp.where(z >= 0.0, z, a_ref[0, 0] * z)


def _tc_layer3(s_ref, c_ref, h_ref, wl_ref, wr_ref, b_ref, wc_ref, bc_ref,
               o_ref):
  cnt = jnp.maximum(c_ref[0][:, 0:1] + c_ref[1][:, 0:1], 1.0)
  mean = (s_ref[0] + s_ref[1]) / cnt
  z = (jnp.dot(mean, wl_ref[...], preferred_element_type=jnp.float32)
       + jnp.dot(h_ref[...], wr_ref[...], preferred_element_type=jnp.float32)
       + b_ref[...])
  o_ref[...] = (jnp.dot(z, wc_ref[...], preferred_element_type=jnp.float32)
                + bc_ref[...])


def _row_spec(d):
  return pl.BlockSpec((_ROWS, d), lambda i: (i, 0))


def _sum_spec(d):
  return pl.BlockSpec((NC, _ROWS, d), lambda i: (0, i, 0))


def _full_spec(r, c):
  return pl.BlockSpec((r, c), lambda i: (0, 0))


@jax.jit
def kernel(x, edge_index, W1l, b1, W1r, W2l, b2, W2r, W3l, b3, W3r, Wc, bc, a):
  src = edge_index[0].astype(jnp.int32)
  dst = edge_index[1].astype(jnp.int32)
  pi = jnp.arange(EPAD, dtype=jnp.int32)
  src_p = jnp.concatenate([src, pi % N]).reshape(NW, NCHUNK, CH)
  dst_p = jnp.concatenate([dst, N + pi % (ACC - N)]).reshape(NW, NCHUNK, CH)

  zrow128 = jnp.zeros((CH, 128), jnp.float32)
  ones1 = jnp.ones((CH, CW), jnp.float32)

  w1l_t = W1l.T
  w1r_t = W1r.T
  w2l_t = W2l.T
  w2r_t = W2r.T
  w3l_t = W3l.T
  w3r_t = W3r.T
  wc_t = Wc.T
  b1_r = b1.reshape(1, -1)
  b2_r = b2.reshape(1, -1)
  b3_r = b3.reshape(1, -1)
  bc_r = bc.reshape(1, -1)
  a_r = a.reshape(1, 1)

  cnt = _counts(dst_p, ones1, zrow128)[:, :, :1]

  s1 = _seg_sum_128(x, src_p, dst_p, zrow128)
  h1 = pl.pallas_call(
      _tc_layer1,
      grid=(_GRID,),
      in_specs=[_sum_spec(128), _sum_spec(1), _row_spec(128),
                _full_spec(128, 128), _full_spec(128, 128),
                _full_spec(1, 128), _full_spec(1, 1)],
      out_specs=_row_spec(128),
      out_shape=jax.ShapeDtypeStruct((N, 128), jnp.float32),
  )(s1, cnt, x, w1l_t, w1r_t, b1_r, a_r)

  s2 = _seg_sum_128(h1, src_p, dst_p, zrow128)
  h2 = pl.pallas_call(
      _tc_layer2,
      grid=(_GRID,),
      in_specs=[_sum_spec(128), _sum_spec(1), _row_spec(128),
                _full_spec(128, 128), _full_spec(128, 128),
                _full_spec(1, 128), _full_spec(1, 1)],
      out_specs=_row_spec(128),
      out_shape=jax.ShapeDtypeStruct((N, 128), jnp.float32),
  )(s2, cnt, h1, w2l_t, w2r_t, b2_r, a_r)

  s3 = _seg_sum_128(h2, src_p, dst_p, zrow128)
  out = pl.pallas_call(
      _tc_layer3,
      grid=(_GRID,),
      in_specs=[_sum_spec(128), _sum_spec(1), _row_spec(128),
                _full_spec(128, 64), _full_spec(128, 64), _full_spec(1, 64),
                _full_spec(64, 4), _full_spec(1, 4)],
      out_specs=_row_spec(4),
      out_shape=jax.ShapeDtypeStruct((N, 4), jnp.float32),
  )(s3, cnt, h2, w3l_t, w3r_t, b3_r, wc_t, bc_r)
  return out


# speedup vs baseline: 9.3936x; 1.2415x over previous
import functools

import jax
import jax.numpy as jnp
from jax import lax
from jax.experimental import pallas as pl
from jax.experimental.pallas import tpu as pltpu
from jax.experimental.pallas import tpu_sc as plsc

N = 10000
E = 320000
NC, NS = 2, 16
NW = NC * NS
CH = 128
EPT = 10240
NCHUNK = EPT // CH
ACC = 10240
RPT = ACC // NS
EPAD = NW * EPT - E

_MESH = dict(core_axis_name="c", subcore_axis_name="s",
             num_cores=NC, num_subcores=NS)


def _make_seg_sum(d, pipelined=True):
  mesh = plsc.VectorSubcoreMesh(**_MESH)
  scratch = [
      pltpu.VMEM((NCHUNK, CH), jnp.int32),
      pltpu.VMEM((NCHUNK, CH), jnp.int32),
      pltpu.VMEM((CH, d), jnp.float32),
      pltpu.VMEM((CH, d), jnp.float32),
      pltpu.VMEM_SHARED((ACC, d), jnp.float32),
      pltpu.SemaphoreType.DMA,
      pltpu.SemaphoreType.DMA,
  ]

  def body(m_hbm, src_hbm, dst_hbm, zrow_hbm, out_hbm,
           sidx, didx, buf_a, buf_b, acc, sem_a, sem_b):
    c = lax.axis_index("c")
    s = lax.axis_index("s")
    wid = c * NS + s

    pltpu.sync_copy(src_hbm.at[wid], sidx)
    pltpu.sync_copy(dst_hbm.at[wid], didx)

    pltpu.sync_copy(zrow_hbm, buf_a)
    for k in range(RPT // CH):
      pltpu.sync_copy(buf_a, acc.at[pl.ds(s * RPT + k * CH, CH)])
    plsc.subcore_barrier()

    if pipelined:
      dummy = m_hbm.at[pl.ds(0, CH)]
      pltpu.async_copy(m_hbm.at[sidx.at[0]], buf_a, sem_a)

      @pl.loop(0, NCHUNK, step=2)
      def _(j):
        pltpu.make_async_copy(dummy, buf_a, sem_a).wait()
        pltpu.async_copy(m_hbm.at[sidx.at[j + 1]], buf_b, sem_b)
        pltpu.sync_copy(buf_a, acc.at[didx.at[j]], add=True)
        pltpu.make_async_copy(dummy, buf_b, sem_b).wait()

        @pl.when(j + 2 < NCHUNK)
        def _():
          pltpu.async_copy(m_hbm.at[sidx.at[j + 2]], buf_a, sem_a)

        pltpu.sync_copy(buf_b, acc.at[didx.at[j + 1]], add=True)
    else:
      @pl.loop(0, NCHUNK)
      def _(j):
        pltpu.async_copy(m_hbm.at[sidx.at[j]], buf_a, sem_a).wait()
        pltpu.sync_copy(buf_a, acc.at[didx.at[j]], add=True)

    plsc.subcore_barrier()
    pltpu.sync_copy(acc.at[pl.ds(s * RPT, RPT)],
                    out_hbm.at[c, pl.ds(s * RPT, RPT)])

  return pl.kernel(body,
                   out_type=jax.ShapeDtypeStruct((NC, ACC, d), jnp.float32),
                   mesh=mesh, scratch_types=scratch)


NCH2 = NCHUNK // 2


def _make_seg_sum_half(d):
  mesh = plsc.VectorSubcoreMesh(**_MESH)
  scratch = [
      pltpu.VMEM((NCH2, CH), jnp.int32),
      pltpu.VMEM((NCH2, CH), jnp.int32),
      pltpu.VMEM((CH, d), jnp.float32),
      pltpu.VMEM((CH, d), jnp.float32),
      pltpu.VMEM_SHARED((ACC, d), jnp.float32),
      pltpu.SemaphoreType.DMA,
      pltpu.SemaphoreType.DMA,
  ]

  def body(m_hbm, src_hbm, dst_hbm, zrow_hbm, out_hbm,
           sidx, didx, buf_a, buf_b, acc, sem_a, sem_b):
    c = lax.axis_index("c")
    s = lax.axis_index("s")
    wid = c * NS + s

    pltpu.sync_copy(src_hbm.at[wid], sidx)
    pltpu.sync_copy(dst_hbm.at[wid], didx)

    pltpu.sync_copy(zrow_hbm, buf_a)
    for k in range(RPT // CH):
      pltpu.sync_copy(buf_a, acc.at[pl.ds(s * RPT + k * CH, CH)])
    plsc.subcore_barrier()

    dummy = m_hbm.at[pl.ds(0, CH)]
    pltpu.async_copy(m_hbm.at[sidx.at[0]], buf_a, sem_a)
    pltpu.async_copy(m_hbm.at[sidx.at[1]], buf_b, sem_b)

    @pl.loop(0, NCH2 - 2, step=2)
    def _(j):
      pltpu.make_async_copy(dummy, buf_a, sem_a).wait()
      pltpu.sync_copy(buf_a, acc.at[didx.at[j]], add=True)
      pltpu.async_copy(m_hbm.at[sidx.at[j + 2]], buf_a, sem_a)
      pltpu.make_async_copy(dummy, buf_b, sem_b).wait()
      pltpu.sync_copy(buf_b, acc.at[didx.at[j + 1]], add=True)
      pltpu.async_copy(m_hbm.at[sidx.at[j + 3]], buf_b, sem_b)

    pltpu.make_async_copy(dummy, buf_a, sem_a).wait()
    pltpu.sync_copy(buf_a, acc.at[didx.at[NCH2 - 2]], add=True)
    pltpu.make_async_copy(dummy, buf_b, sem_b).wait()
    pltpu.sync_copy(buf_b, acc.at[didx.at[NCH2 - 1]], add=True)

    plsc.subcore_barrier()
    pltpu.sync_copy(acc.at[pl.ds(s * RPT, RPT)],
                    out_hbm.at[c, pl.ds(s * RPT, RPT)])

  return pl.kernel(body,
                   out_type=jax.ShapeDtypeStruct((NC, ACC, d), jnp.float32),
                   mesh=mesh, scratch_types=scratch)


CW = 128


def _make_counts():
  mesh = plsc.VectorSubcoreMesh(**_MESH)
  scratch = [
      pltpu.VMEM((NCHUNK, CH), jnp.int32),
      pltpu.VMEM((CH, CW), jnp.float32),
      pltpu.VMEM_SHARED((ACC, CW), jnp.float32),
  ]

  def body(dst_hbm, ones_hbm, zrow_hbm, cnt_hbm, didx, buf, cacc):
    c = lax.axis_index("c")
    s = lax.axis_index("s")
    wid = c * NS + s
    pltpu.sync_copy(dst_hbm.at[wid], didx)
    pltpu.sync_copy(zrow_hbm, buf)
    for k in range(RPT // CH):
      pltpu.sync_copy(buf, cacc.at[pl.ds(s * RPT + k * CH, CH)])
    pltpu.sync_copy(ones_hbm, buf)
    plsc.subcore_barrier()

    @pl.loop(0, NCHUNK)
    def _(j):
      pltpu.sync_copy(buf, cacc.at[didx.at[j]], add=True)

    plsc.subcore_barrier()
    pltpu.sync_copy(cacc.at[pl.ds(s * RPT, RPT)],
                    cnt_hbm.at[c, pl.ds(s * RPT, RPT)])

  return pl.kernel(body,
                   out_type=jax.ShapeDtypeStruct((NC, ACC, CW), jnp.float32),
                   mesh=mesh, scratch_types=scratch)


_seg_sum_128 = _make_seg_sum(128, pipelined=False)
_seg_sum_half_128 = _make_seg_sum_half(128)
_counts = _make_counts()

_ROWS = 400
_GRID = N // _ROWS


def _tc_layer1(sa_ref, sb_ref, c_ref, x_ref, wl_ref, wr_ref, b_ref, a_ref,
               o_ref):
  cnt = jnp.maximum(c_ref[0][:, 0:1] + c_ref[1][:, 0:1], 1.0)
  mean = (sa_ref[0] + sa_ref[1] + sb_ref[0] + sb_ref[1]) / cnt
  z = (jnp.dot(mean, wl_ref[...], preferred_element_type=jnp.float32)
       + jnp.dot(x_ref[...], wr_ref[...], preferred_element_type=jnp.float32)
       + b_ref[...])
  o_ref[...] = jnp.where(z >= 0.0, z, a_ref[0, 0] * z)


def _tc_layer2(sa_ref, sb_ref, c_ref, h_ref, wl_ref, wr_ref, b_ref, a_ref,
               o_ref):
  cnt = jnp.maximum(c_ref[0][:, 0:1] + c_ref[1][:, 0:1], 1.0)
  mean = (sa_ref[0] + sa_ref[1] + sb_ref[0] + sb_ref[1]) / cnt
  z = (jnp.dot(mean, wl_ref[...], preferred_element_type=jnp.float32)
       + jnp.dot(h_ref[...], wr_ref[...], preferred_element_type=jnp.float32)
       + b_ref[...])
  o_ref[...] = jnp.where(z >= 0.0, z, a_ref[0, 0] * z)


def _tc_layer3(sa_ref, sb_ref, c_ref, h_ref, wl_ref, wr_ref, b_ref, wc_ref,
               bc_ref, o_ref):
  cnt = jnp.maximum(c_ref[0][:, 0:1] + c_ref[1][:, 0:1], 1.0)
  mean = (sa_ref[0] + sa_ref[1] + sb_ref[0] + sb_ref[1]) / cnt
  z = (jnp.dot(mean, wl_ref[...], preferred_element_type=jnp.float32)
       + jnp.dot(h_ref[...], wr_ref[...], preferred_element_type=jnp.float32)
       + b_ref[...])
  o_ref[...] = (jnp.dot(z, wc_ref[...], preferred_element_type=jnp.float32)
                + bc_ref[...])


def _row_spec(d):
  return pl.BlockSpec((_ROWS, d), lambda i: (i, 0))


def _sum_spec(d):
  return pl.BlockSpec((NC, _ROWS, d), lambda i: (0, i, 0))


def _full_spec(r, c):
  return pl.BlockSpec((r, c), lambda i: (0, 0))


@jax.jit
def kernel(x, edge_index, W1l, b1, W1r, W2l, b2, W2r, W3l, b3, W3r, Wc, bc, a):
  src = edge_index[0].astype(jnp.int32)
  dst = edge_index[1].astype(jnp.int32)
  pi = jnp.arange(EPAD, dtype=jnp.int32)
  src_p = jnp.concatenate([src, pi % N]).reshape(NW, NCHUNK, CH)
  dst_p = jnp.concatenate([dst, N + pi % (ACC - N)]).reshape(NW, NCHUNK, CH)
  src_a = src_p[:, :NCH2]
  src_b = src_p[:, NCH2:]
  dst_a = dst_p[:, :NCH2]
  dst_b = dst_p[:, NCH2:]

  zrow128 = jnp.zeros((CH, 128), jnp.float32)
  ones1 = jnp.ones((CH, CW), jnp.float32)

  w1l_t = W1l.T
  w1r_t = W1r.T
  w2l_t = W2l.T
  w2r_t = W2r.T
  w3l_t = W3l.T
  w3r_t = W3r.T
  wc_t = Wc.T
  b1_r = b1.reshape(1, -1)
  b2_r = b2.reshape(1, -1)
  b3_r = b3.reshape(1, -1)
  bc_r = bc.reshape(1, -1)
  a_r = a.reshape(1, 1)

  cnt = _counts(dst_p, ones1, zrow128)[:, :, :1]

  s1a = _seg_sum_half_128(x, src_a, dst_a, zrow128)
  s1b = _seg_sum_half_128(x, src_b, dst_b, zrow128)
  h1 = pl.pallas_call(
      _tc_layer1,
      grid=(_GRID,),
      in_specs=[_sum_spec(128), _sum_spec(128), _sum_spec(1), _row_spec(128),
                _full_spec(128, 128), _full_spec(128, 128),
                _full_spec(1, 128), _full_spec(1, 1)],
      out_specs=_row_spec(128),
      out_shape=jax.ShapeDtypeStruct((N, 128), jnp.float32),
  )(s1a, s1b, cnt, x, w1l_t, w1r_t, b1_r, a_r)

  s2a = _seg_sum_half_128(h1, src_a, dst_a, zrow128)
  s2b = _seg_sum_half_128(h1, src_b, dst_b, zrow128)
  h2 = pl.pallas_call(
      _tc_layer2,
      grid=(_GRID,),
      in_specs=[_sum_spec(128), _sum_spec(128), _sum_spec(1), _row_spec(128),
                _full_spec(128, 128), _full_spec(128, 128),
                _full_spec(1, 128), _full_spec(1, 1)],
      out_specs=_row_spec(128),
      out_shape=jax.ShapeDtypeStruct((N, 128), jnp.float32),
  )(s2a, s2b, cnt, h1, w2l_t, w2r_t, b2_r, a_r)

  s3a = _seg_sum_half_128(h2, src_a, dst_a, zrow128)
  s3b = _seg_sum_half_128(h2, src_b, dst_b, zrow128)
  out = pl.pallas_call(
      _tc_layer3,
      grid=(_GRID,),
      in_specs=[_sum_spec(128), _sum_spec(128), _sum_spec(1), _row_spec(128),
                _full_spec(128, 64), _full_spec(128, 64), _full_spec(1, 64),
                _full_spec(64, 4), _full_spec(1, 4)],
      out_specs=_row_spec(4),
      out_shape=jax.ShapeDtypeStruct((N, 4), jnp.float32),
  )(s3a, s3b, cnt, h2, w3l_t, w3r_t, b3_r, wc_t, bc_r)
  return out


# speedup vs baseline: 9.4303x; 1.0039x over previous
import functools

import jax
import jax.numpy as jnp
from jax import lax
from jax.experimental import pallas as pl
from jax.experimental.pallas import tpu as pltpu
from jax.experimental.pallas import tpu_sc as plsc

N = 10000
E = 320000
NC, NS = 2, 16
NW = NC * NS
CH = 128
EPT = 10240
NCHUNK = EPT // CH
ACC = 10240
RPT = ACC // NS
EPAD = NW * EPT - E

_MESH = dict(core_axis_name="c", subcore_axis_name="s",
             num_cores=NC, num_subcores=NS)


def _make_seg_sum(d, pipelined=True):
  mesh = plsc.VectorSubcoreMesh(**_MESH)
  scratch = [
      pltpu.VMEM((NCHUNK, CH), jnp.int32),
      pltpu.VMEM((NCHUNK, CH), jnp.int32),
      pltpu.VMEM((CH, d), jnp.float32),
      pltpu.VMEM((CH, d), jnp.float32),
      pltpu.VMEM_SHARED((ACC, d), jnp.float32),
      pltpu.SemaphoreType.DMA,
      pltpu.SemaphoreType.DMA,
  ]

  def body(m_hbm, src_hbm, dst_hbm, zrow_hbm, out_hbm,
           sidx, didx, buf_a, buf_b, acc, sem_a, sem_b):
    c = lax.axis_index("c")
    s = lax.axis_index("s")
    wid = c * NS + s

    pltpu.sync_copy(src_hbm.at[wid], sidx)
    pltpu.sync_copy(dst_hbm.at[wid], didx)

    pltpu.sync_copy(zrow_hbm, buf_a)
    for k in range(RPT // CH):
      pltpu.sync_copy(buf_a, acc.at[pl.ds(s * RPT + k * CH, CH)])
    plsc.subcore_barrier()

    if pipelined:
      dummy = m_hbm.at[pl.ds(0, CH)]
      pltpu.async_copy(m_hbm.at[sidx.at[0]], buf_a, sem_a)

      @pl.loop(0, NCHUNK, step=2)
      def _(j):
        pltpu.make_async_copy(dummy, buf_a, sem_a).wait()
        pltpu.async_copy(m_hbm.at[sidx.at[j + 1]], buf_b, sem_b)
        pltpu.sync_copy(buf_a, acc.at[didx.at[j]], add=True)
        pltpu.make_async_copy(dummy, buf_b, sem_b).wait()

        @pl.when(j + 2 < NCHUNK)
        def _():
          pltpu.async_copy(m_hbm.at[sidx.at[j + 2]], buf_a, sem_a)

        pltpu.sync_copy(buf_b, acc.at[didx.at[j + 1]], add=True)
    else:
      @pl.loop(0, NCHUNK)
      def _(j):
        pltpu.async_copy(m_hbm.at[sidx.at[j]], buf_a, sem_a).wait()
        pltpu.sync_copy(buf_a, acc.at[didx.at[j]], add=True)

    plsc.subcore_barrier()
    pltpu.sync_copy(acc.at[pl.ds(s * RPT, RPT)],
                    out_hbm.at[c, pl.ds(s * RPT, RPT)])

  return pl.kernel(body,
                   out_type=jax.ShapeDtypeStruct((NC, ACC, d), jnp.float32),
                   mesh=mesh, scratch_types=scratch)


NCH2 = NCHUNK // 2


def _make_seg_sum_half(d):
  mesh = plsc.VectorSubcoreMesh(**_MESH)
  scratch = [
      pltpu.VMEM((NCH2, CH), jnp.int32),
      pltpu.VMEM((NCH2, CH), jnp.int32),
      pltpu.VMEM((CH, d), jnp.float32),
      pltpu.VMEM((CH, d), jnp.float32),
      pltpu.VMEM_SHARED((ACC, d), jnp.float32),
      pltpu.SemaphoreType.DMA,
      pltpu.SemaphoreType.DMA,
  ]

  def body(m_hbm, src_hbm, dst_hbm, zrow_hbm, out_hbm,
           sidx, didx, buf_a, buf_b, acc, sem_a, sem_b):
    c = lax.axis_index("c")
    s = lax.axis_index("s")
    wid = c * NS + s

    pltpu.sync_copy(src_hbm.at[wid], sidx)
    pltpu.sync_copy(dst_hbm.at[wid], didx)

    pltpu.async_copy(m_hbm.at[sidx.at[0]], buf_a, sem_a)
    pltpu.sync_copy(zrow_hbm, buf_b)
    for k in range(RPT // CH):
      pltpu.sync_copy(buf_b, acc.at[pl.ds(s * RPT + k * CH, CH)])
    pltpu.async_copy(m_hbm.at[sidx.at[1]], buf_b, sem_b)
    plsc.subcore_barrier()

    dummy = m_hbm.at[pl.ds(0, CH)]

    @pl.loop(0, NCH2 - 2, step=2)
    def _(j):
      pltpu.make_async_copy(dummy, buf_a, sem_a).wait()
      pltpu.sync_copy(buf_a, acc.at[didx.at[j]], add=True)
      pltpu.async_copy(m_hbm.at[sidx.at[j + 2]], buf_a, sem_a)
      pltpu.make_async_copy(dummy, buf_b, sem_b).wait()
      pltpu.sync_copy(buf_b, acc.at[didx.at[j + 1]], add=True)
      pltpu.async_copy(m_hbm.at[sidx.at[j + 3]], buf_b, sem_b)

    pltpu.make_async_copy(dummy, buf_a, sem_a).wait()
    pltpu.sync_copy(buf_a, acc.at[didx.at[NCH2 - 2]], add=True)
    pltpu.make_async_copy(dummy, buf_b, sem_b).wait()
    pltpu.sync_copy(buf_b, acc.at[didx.at[NCH2 - 1]], add=True)

    plsc.subcore_barrier()
    pltpu.sync_copy(acc.at[pl.ds(s * RPT, RPT)],
                    out_hbm.at[c, pl.ds(s * RPT, RPT)])

  return pl.kernel(body,
                   out_type=jax.ShapeDtypeStruct((NC, ACC, d), jnp.float32),
                   mesh=mesh, scratch_types=scratch)


CW = 128


def _make_counts():
  mesh = plsc.VectorSubcoreMesh(**_MESH)
  scratch = [
      pltpu.VMEM((NCHUNK, CH), jnp.int32),
      pltpu.VMEM((CH, CW), jnp.float32),
      pltpu.VMEM_SHARED((ACC, CW), jnp.float32),
  ]

  def body(dst_hbm, ones_hbm, zrow_hbm, cnt_hbm, didx, buf, cacc):
    c = lax.axis_index("c")
    s = lax.axis_index("s")
    wid = c * NS + s
    pltpu.sync_copy(dst_hbm.at[wid], didx)
    pltpu.sync_copy(zrow_hbm, buf)
    for k in range(RPT // CH):
      pltpu.sync_copy(buf, cacc.at[pl.ds(s * RPT + k * CH, CH)])
    pltpu.sync_copy(ones_hbm, buf)
    plsc.subcore_barrier()

    @pl.loop(0, NCHUNK)
    def _(j):
      pltpu.sync_copy(buf, cacc.at[didx.at[j]], add=True)

    plsc.subcore_barrier()
    pltpu.sync_copy(cacc.at[pl.ds(s * RPT, RPT)],
                    cnt_hbm.at[c, pl.ds(s * RPT, RPT)])

  return pl.kernel(body,
                   out_type=jax.ShapeDtypeStruct((NC, ACC, CW), jnp.float32),
                   mesh=mesh, scratch_types=scratch)


_seg_sum_128 = _make_seg_sum(128, pipelined=False)
_seg_sum_half_128 = _make_seg_sum_half(128)
_counts = _make_counts()

_ROWS = 400
_GRID = N // _ROWS


def _tc_layer1(sa_ref, sb_ref, c_ref, x_ref, wl_ref, wr_ref, b_ref, a_ref,
               o_ref):
  cnt = jnp.maximum(c_ref[0][:, 0:1] + c_ref[1][:, 0:1], 1.0)
  mean = (sa_ref[0] + sa_ref[1] + sb_ref[0] + sb_ref[1]) / cnt
  z = (jnp.dot(mean, wl_ref[...], preferred_element_type=jnp.float32)
       + jnp.dot(x_ref[...], wr_ref[...], preferred_element_type=jnp.float32)
       + b_ref[...])
  o_ref[...] = jnp.where(z >= 0.0, z, a_ref[0, 0] * z)


def _tc_layer2(sa_ref, sb_ref, c_ref, h_ref, wl_ref, wr_ref, b_ref, a_ref,
               o_ref):
  cnt = jnp.maximum(c_ref[0][:, 0:1] + c_ref[1][:, 0:1], 1.0)
  mean = (sa_ref[0] + sa_ref[1] + sb_ref[0] + sb_ref[1]) / cnt
  z = (jnp.dot(mean, wl_ref[...], preferred_element_type=jnp.float32)
       + jnp.dot(h_ref[...], wr_ref[...], preferred_element_type=jnp.float32)
       + b_ref[...])
  o_ref[...] = jnp.where(z >= 0.0, z, a_ref[0, 0] * z)


def _tc_layer3(sa_ref, sb_ref, c_ref, h_ref, wl_ref, wr_ref, b_ref, wc_ref,
               bc_ref, o_ref):
  cnt = jnp.maximum(c_ref[0][:, 0:1] + c_ref[1][:, 0:1], 1.0)
  mean = (sa_ref[0] + sa_ref[1] + sb_ref[0] + sb_ref[1]) / cnt
  z = (jnp.dot(mean, wl_ref[...], preferred_element_type=jnp.float32)
       + jnp.dot(h_ref[...], wr_ref[...], preferred_element_type=jnp.float32)
       + b_ref[...])
  o_ref[...] = (jnp.dot(z, wc_ref[...], preferred_element_type=jnp.float32)
                + bc_ref[...])


def _row_spec(d):
  return pl.BlockSpec((_ROWS, d), lambda i: (i, 0))


def _sum_spec(d):
  return pl.BlockSpec((NC, _ROWS, d), lambda i: (0, i, 0))


def _full_spec(r, c):
  return pl.BlockSpec((r, c), lambda i: (0, 0))


@jax.jit
def kernel(x, edge_index, W1l, b1, W1r, W2l, b2, W2r, W3l, b3, W3r, Wc, bc, a):
  src = edge_index[0].astype(jnp.int32)
  dst = edge_index[1].astype(jnp.int32)
  pi = jnp.arange(EPAD, dtype=jnp.int32)
  src_p = jnp.concatenate([src, pi % N]).reshape(NW, NCHUNK, CH)
  dst_p = jnp.concatenate([dst, N + pi % (ACC - N)]).reshape(NW, NCHUNK, CH)
  src_a = src_p[:, :NCH2]
  src_b = src_p[:, NCH2:]
  dst_a = dst_p[:, :NCH2]
  dst_b = dst_p[:, NCH2:]

  zrow128 = jnp.zeros((CH, 128), jnp.float32)
  ones1 = jnp.ones((CH, CW), jnp.float32)

  w1l_t = W1l.T
  w1r_t = W1r.T
  w2l_t = W2l.T
  w2r_t = W2r.T
  w3l_t = W3l.T
  w3r_t = W3r.T
  wc_t = Wc.T
  b1_r = b1.reshape(1, -1)
  b2_r = b2.reshape(1, -1)
  b3_r = b3.reshape(1, -1)
  bc_r = bc.reshape(1, -1)
  a_r = a.reshape(1, 1)

  cnt = _counts(dst_p, ones1, zrow128)[:, :, :1]

  s1a = _seg_sum_half_128(x, src_a, dst_a, zrow128)
  s1b = _seg_sum_half_128(x, src_b, dst_b, zrow128)
  h1 = pl.pallas_call(
      _tc_layer1,
      grid=(_GRID,),
      in_specs=[_sum_spec(128), _sum_spec(128), _sum_spec(1), _row_spec(128),
                _full_spec(128, 128), _full_spec(128, 128),
                _full_spec(1, 128), _full_spec(1, 1)],
      out_specs=_row_spec(128),
      out_shape=jax.ShapeDtypeStruct((N, 128), jnp.float32),
  )(s1a, s1b, cnt, x, w1l_t, w1r_t, b1_r, a_r)

  s2a = _seg_sum_half_128(h1, src_a, dst_a, zrow128)
  s2b = _seg_sum_half_128(h1, src_b, dst_b, zrow128)
  h2 = pl.pallas_call(
      _tc_layer2,
      grid=(_GRID,),
      in_specs=[_sum_spec(128), _sum_spec(128), _sum_spec(1), _row_spec(128),
                _full_spec(128, 128), _full_spec(128, 128),
                _full_spec(1, 128), _full_spec(1, 1)],
      out_specs=_row_spec(128),
      out_shape=jax.ShapeDtypeStruct((N, 128), jnp.float32),
  )(s2a, s2b, cnt, h1, w2l_t, w2r_t, b2_r, a_r)

  s3a = _seg_sum_half_128(h2, src_a, dst_a, zrow128)
  s3b = _seg_sum_half_128(h2, src_b, dst_b, zrow128)
  out = pl.pallas_call(
      _tc_layer3,
      grid=(_GRID,),
      in_specs=[_sum_spec(128), _sum_spec(128), _sum_spec(1), _row_spec(128),
                _full_spec(128, 64), _full_spec(128, 64), _full_spec(1, 64),
                _full_spec(64, 4), _full_spec(1, 4)],
      out_specs=_row_spec(4),
      out_shape=jax.ShapeDtypeStruct((N, 4), jnp.float32),
  )(s3a, s3b, cnt, h2, w3l_t, w3r_t, b3_r, wc_t, bc_r)
  return out


# speedup vs baseline: 9.4700x; 1.0042x over previous
import jax
import jax.numpy as jnp
from jax import lax
from jax.experimental import pallas as pl
from jax.experimental.pallas import tpu as pltpu
from jax.experimental.pallas import tpu_sc as plsc

N = 10000
E = 320000
NC, NS = 2, 16
NW = NC * NS
CH = 128
EPT = 10240
NCHUNK = EPT // CH
ACC = 10240
RPT = ACC // NS
EPAD = NW * EPT - E

_MESH = dict(core_axis_name="c", subcore_axis_name="s",
             num_cores=NC, num_subcores=NS)


NCH2 = NCHUNK // 2


def _make_seg_sum_half(d):
  mesh = plsc.VectorSubcoreMesh(**_MESH)
  scratch = [
      pltpu.VMEM((NCH2, CH), jnp.int32),
      pltpu.VMEM((NCH2, CH), jnp.int32),
      pltpu.VMEM((CH, d), jnp.float32),
      pltpu.VMEM((CH, d), jnp.float32),
      pltpu.VMEM_SHARED((ACC, d), jnp.float32),
      pltpu.SemaphoreType.DMA,
      pltpu.SemaphoreType.DMA,
  ]

  def body(m_hbm, src_hbm, dst_hbm, zrow_hbm, out_hbm,
           sidx, didx, buf_a, buf_b, acc, sem_a, sem_b):
    c = lax.axis_index("c")
    s = lax.axis_index("s")
    wid = c * NS + s

    pltpu.sync_copy(src_hbm.at[wid], sidx)
    pltpu.sync_copy(dst_hbm.at[wid], didx)

    pltpu.async_copy(m_hbm.at[sidx.at[0]], buf_a, sem_a)
    pltpu.sync_copy(zrow_hbm, buf_b)
    for k in range(RPT // CH):
      pltpu.sync_copy(buf_b, acc.at[pl.ds(s * RPT + k * CH, CH)])
    pltpu.async_copy(m_hbm.at[sidx.at[1]], buf_b, sem_b)
    plsc.subcore_barrier()

    dummy = m_hbm.at[pl.ds(0, CH)]

    @pl.loop(0, NCH2 - 2, step=2)
    def _(j):
      pltpu.make_async_copy(dummy, buf_a, sem_a).wait()
      pltpu.sync_copy(buf_a, acc.at[didx.at[j]], add=True)
      pltpu.async_copy(m_hbm.at[sidx.at[j + 2]], buf_a, sem_a)
      pltpu.make_async_copy(dummy, buf_b, sem_b).wait()
      pltpu.sync_copy(buf_b, acc.at[didx.at[j + 1]], add=True)
      pltpu.async_copy(m_hbm.at[sidx.at[j + 3]], buf_b, sem_b)

    pltpu.make_async_copy(dummy, buf_a, sem_a).wait()
    pltpu.sync_copy(buf_a, acc.at[didx.at[NCH2 - 2]], add=True)
    pltpu.make_async_copy(dummy, buf_b, sem_b).wait()
    pltpu.sync_copy(buf_b, acc.at[didx.at[NCH2 - 1]], add=True)

    plsc.subcore_barrier()
    pltpu.sync_copy(acc.at[pl.ds(s * RPT, RPT)],
                    out_hbm.at[c, pl.ds(s * RPT, RPT)])

  return pl.kernel(body,
                   out_type=jax.ShapeDtypeStruct((NC, ACC, d), jnp.float32),
                   mesh=mesh, scratch_types=scratch)


CW = 128


def _make_counts():
  mesh = plsc.VectorSubcoreMesh(**_MESH)
  scratch = [
      pltpu.VMEM((NCHUNK, CH), jnp.int32),
      pltpu.VMEM((CH, CW), jnp.float32),
      pltpu.VMEM_SHARED((ACC, CW), jnp.float32),
  ]

  def body(dst_hbm, ones_hbm, zrow_hbm, cnt_hbm, didx, buf, cacc):
    c = lax.axis_index("c")
    s = lax.axis_index("s")
    wid = c * NS + s
    pltpu.sync_copy(dst_hbm.at[wid], didx)
    pltpu.sync_copy(zrow_hbm, buf)
    for k in range(RPT // CH):
      pltpu.sync_copy(buf, cacc.at[pl.ds(s * RPT + k * CH, CH)])
    pltpu.sync_copy(ones_hbm, buf)
    plsc.subcore_barrier()

    @pl.loop(0, NCHUNK)
    def _(j):
      pltpu.sync_copy(buf, cacc.at[didx.at[j]], add=True)

    plsc.subcore_barrier()
    pltpu.sync_copy(cacc.at[pl.ds(s * RPT, RPT)],
                    cnt_hbm.at[c, pl.ds(s * RPT, RPT)])

  return pl.kernel(body,
                   out_type=jax.ShapeDtypeStruct((NC, ACC, CW), jnp.float32),
                   mesh=mesh, scratch_types=scratch)


_seg_sum_half_128 = _make_seg_sum_half(128)
_counts = _make_counts()

_ROWS = 400
_GRID = N // _ROWS


def _tc_layer1(sa_ref, sb_ref, c_ref, x_ref, wl_ref, wr_ref, b_ref, a_ref,
               o_ref):
  cnt = jnp.maximum(c_ref[0][:, 0:1] + c_ref[1][:, 0:1], 1.0)
  mean = (sa_ref[0] + sa_ref[1] + sb_ref[0] + sb_ref[1]) / cnt
  z = (jnp.dot(mean, wl_ref[...], preferred_element_type=jnp.float32)
       + jnp.dot(x_ref[...], wr_ref[...], preferred_element_type=jnp.float32)
       + b_ref[...])
  o_ref[...] = jnp.where(z >= 0.0, z, a_ref[0, 0] * z)


def _tc_layer2(sa_ref, sb_ref, c_ref, h_ref, wl_ref, wr_ref, b_ref, a_ref,
               o_ref):
  cnt = jnp.maximum(c_ref[0][:, 0:1] + c_ref[1][:, 0:1], 1.0)
  mean = (sa_ref[0] + sa_ref[1] + sb_ref[0] + sb_ref[1]) / cnt
  z = (jnp.dot(mean, wl_ref[...], preferred_element_type=jnp.float32)
       + jnp.dot(h_ref[...], wr_ref[...], preferred_element_type=jnp.float32)
       + b_ref[...])
  o_ref[...] = jnp.where(z >= 0.0, z, a_ref[0, 0] * z)


def _tc_layer3(sa_ref, sb_ref, c_ref, h_ref, wl_ref, wr_ref, b_ref, wc_ref,
               bc_ref, o_ref):
  cnt = jnp.maximum(c_ref[0][:, 0:1] + c_ref[1][:, 0:1], 1.0)
  mean = (sa_ref[0] + sa_ref[1] + sb_ref[0] + sb_ref[1]) / cnt
  z = (jnp.dot(mean, wl_ref[...], preferred_element_type=jnp.float32)
       + jnp.dot(h_ref[...], wr_ref[...], preferred_element_type=jnp.float32)
       + b_ref[...])
  o_ref[...] = (jnp.dot(z, wc_ref[...], preferred_element_type=jnp.float32)
                + bc_ref[...])


def _row_spec(d):
  return pl.BlockSpec((_ROWS, d), lambda i: (i, 0))


def _sum_spec(d):
  return pl.BlockSpec((NC, _ROWS, d), lambda i: (0, i, 0))


def _full_spec(r, c):
  return pl.BlockSpec((r, c), lambda i: (0, 0))


@jax.jit
def kernel(x, edge_index, W1l, b1, W1r, W2l, b2, W2r, W3l, b3, W3r, Wc, bc, a):
  src = edge_index[0].astype(jnp.int32)
  dst = edge_index[1].astype(jnp.int32)
  pi = jnp.arange(EPAD, dtype=jnp.int32)
  src_p = jnp.concatenate([src, pi % N]).reshape(NW, NCHUNK, CH)
  dst_p = jnp.concatenate([dst, N + pi % (ACC - N)]).reshape(NW, NCHUNK, CH)
  src_a = src_p[:, :NCH2]
  src_b = src_p[:, NCH2:]
  dst_a = dst_p[:, :NCH2]
  dst_b = dst_p[:, NCH2:]

  zrow128 = jnp.zeros((CH, 128), jnp.float32)
  ones1 = jnp.ones((CH, CW), jnp.float32)

  w1l_t = W1l.T
  w1r_t = W1r.T
  w2l_t = W2l.T
  w2r_t = W2r.T
  w3l_t = W3l.T
  w3r_t = W3r.T
  wc_t = Wc.T
  b1_r = b1.reshape(1, -1)
  b2_r = b2.reshape(1, -1)
  b3_r = b3.reshape(1, -1)
  bc_r = bc.reshape(1, -1)
  a_r = a.reshape(1, 1)

  cnt = _counts(dst_p, ones1, zrow128)[:, :, :1]

  s1a = _seg_sum_half_128(x, src_a, dst_a, zrow128)
  s1b = _seg_sum_half_128(x, src_b, dst_b, zrow128)
  h1 = pl.pallas_call(
      _tc_layer1,
      grid=(_GRID,),
      in_specs=[_sum_spec(128), _sum_spec(128), _sum_spec(1), _row_spec(128),
                _full_spec(128, 128), _full_spec(128, 128),
                _full_spec(1, 128), _full_spec(1, 1)],
      out_specs=_row_spec(128),
      out_shape=jax.ShapeDtypeStruct((N, 128), jnp.float32),
  )(s1a, s1b, cnt, x, w1l_t, w1r_t, b1_r, a_r)

  s2a = _seg_sum_half_128(h1, src_a, dst_a, zrow128)
  s2b = _seg_sum_half_128(h1, src_b, dst_b, zrow128)
  h2 = pl.pallas_call(
      _tc_layer2,
      grid=(_GRID,),
      in_specs=[_sum_spec(128), _sum_spec(128), _sum_spec(1), _row_spec(128),
                _full_spec(128, 128), _full_spec(128, 128),
                _full_spec(1, 128), _full_spec(1, 1)],
      out_specs=_row_spec(128),
      out_shape=jax.ShapeDtypeStruct((N, 128), jnp.float32),
  )(s2a, s2b, cnt, h1, w2l_t, w2r_t, b2_r, a_r)

  s3a = _seg_sum_half_128(h2, src_a, dst_a, zrow128)
  s3b = _seg_sum_half_128(h2, src_b, dst_b, zrow128)
  out = pl.pallas_call(
      _tc_layer3,
      grid=(_GRID,),
      in_specs=[_sum_spec(128), _sum_spec(128), _sum_spec(1), _row_spec(128),
                _full_spec(128, 64), _full_spec(128, 64), _full_spec(1, 64),
                _full_spec(64, 4), _full_spec(1, 4)],
      out_specs=_row_spec(4),
      out_shape=jax.ShapeDtypeStruct((N, 4), jnp.float32),
  )(s3a, s3b, cnt, h2, w3l_t, w3r_t, b3_r, wc_t, bc_r)
  return out
